# baseline (speedup 1.0000x reference)
import functools
import jax
import jax.numpy as jnp
from jax import lax
from jax.experimental import pallas as pl
from jax.experimental.pallas import tpu as pltpu
from jax.experimental.pallas import tpu_sc as plsc

T = 288
PERIOD = 144
HALF_W = 12
NC, NS, LANES = 2, 16, 16
NW = NC * NS
R = 83200
G_HI, G_LO = 163, 162
ROWS_HI = G_HI * LANES
ROWS_LO = G_LO * LANES
CHUNK = LANES * T


def _sc_body(x_hbm, hf_hbm, da_hbm, tr_hbm, xb, hfb, dab, trb):
    wid = lax.axis_index("s") * NC + lax.axis_index("c")
    base_row = jnp.where(wid < 16, wid * ROWS_HI,
                         16 * ROWS_HI + (wid - 16) * ROWS_LO)
    ngroups = jnp.where(wid < 16, G_HI, G_LO)
    loff = lax.iota(jnp.int32, 16) * T

    def group_body(i, carry):
        base = (base_row + i * LANES) * T
        pltpu.sync_copy(x_hbm.at[pl.ds(base, CHUNK)], xb)

        def g(t):
            return plsc.load_gather(xb, [loff + t])

        def emit(t, w, xt, partner):
            dly = 0.5 * (xt + partner)
            tr = w * (1.0 / 25.0)
            hf = xt - 0.5 * tr - 0.5 * dly
            idx = loff + t
            plsc.store_scatter(trb, [idx], tr)
            plsc.store_scatter(dab, [idx], dly)
            plsc.store_scatter(hfb, [idx], hf)

        def init_s(s, w):
            return w + g(s)

        w = lax.fori_loop(0, HALF_W, init_s, jnp.zeros((16,), jnp.float32))

        def phase(add, sub, poff):
            def body(t, w):
                if add:
                    w = w + g(t + HALF_W)
                if sub:
                    w = w - g(t - HALF_W - 1)
                xt = g(t)
                emit(t, w, xt, g(t + poff))
                return w
            return body

        w = lax.fori_loop(0, HALF_W + 1, phase(True, False, PERIOD), w)
        w = lax.fori_loop(HALF_W + 1, PERIOD, phase(True, True, PERIOD), w)
        w = lax.fori_loop(PERIOD, T - HALF_W, phase(True, True, -PERIOD), w)
        w = lax.fori_loop(T - HALF_W, T, phase(False, True, -PERIOD), w)

        pltpu.sync_copy(hfb, hf_hbm.at[pl.ds(base, CHUNK)])
        pltpu.sync_copy(dab, da_hbm.at[pl.ds(base, CHUNK)])
        pltpu.sync_copy(trb, tr_hbm.at[pl.ds(base, CHUNK)])
        return carry

    lax.fori_loop(0, ngroups, group_body, 0)


@jax.jit
def _run(xf):
    out = jax.ShapeDtypeStruct((R * T,), jnp.float32)
    mesh = plsc.VectorSubcoreMesh(core_axis_name="c", subcore_axis_name="s",
                                  num_cores=NC, num_subcores=NS)
    f = pl.kernel(
        _sc_body,
        out_type=(out, out, out),
        mesh=mesh,
        scratch_types=[
            pltpu.VMEM((CHUNK,), jnp.float32),
            pltpu.VMEM((CHUNK,), jnp.float32),
            pltpu.VMEM((CHUNK,), jnp.float32),
            pltpu.VMEM((CHUNK,), jnp.float32),
        ],
        compiler_params=pltpu.CompilerParams(use_tc_tiling_on_sc=False,
                                             needs_layout_passes=False),
    )
    return f(xf)


def kernel(x):
    shape = x.shape
    hf, da, tr = _run(x.reshape(-1))
    return hf.reshape(shape), da.reshape(shape), tr.reshape(shape)


# speedup vs baseline: 1.0412x; 1.0412x over previous
import jax
import jax.numpy as jnp
from jax import lax
from jax.experimental import pallas as pl
from jax.experimental.pallas import tpu as pltpu
from jax.experimental.pallas import tpu_sc as plsc

T = 288
PERIOD = 144
HALF_W = 12
NC, NS, LANES = 2, 16, 16
NW = NC * NS
R = 83200
G_HI, G_LO = 163, 162
ROWS_HI = G_HI * LANES
ROWS_LO = G_LO * LANES
CHUNK = LANES * T
UNROLL = 8


def _sc_body(x_hbm, hf_hbm, da_hbm, tr_hbm, xb, hfb, dab, trb, in_sems, out_sems):
    wid = lax.axis_index("s") * NC + lax.axis_index("c")
    base_row = jnp.where(wid < 16, wid * ROWS_HI,
                         16 * ROWS_HI + (wid - 16) * ROWS_LO)
    ngroups = jnp.where(wid < 16, G_HI, G_LO)
    loff = lax.iota(jnp.int32, 16) * T

    def ebase(i):
        return (base_row + i * LANES) * T

    def in_copy(i, slot):
        return pltpu.make_async_copy(
            x_hbm.at[pl.ds(ebase(i), CHUNK)], xb.at[slot], in_sems.at[slot])

    def out_copies(i, slot):
        sl = pl.ds(ebase(i), CHUNK)
        return (
            pltpu.make_async_copy(hfb.at[slot], hf_hbm.at[sl], out_sems.at[slot]),
            pltpu.make_async_copy(dab.at[slot], da_hbm.at[sl], out_sems.at[slot]),
            pltpu.make_async_copy(trb.at[slot], tr_hbm.at[sl], out_sems.at[slot]),
        )

    def compute(slot):
        xs = xb.at[slot]

        def g(t):
            return plsc.load_gather(xs, [loff + t])

        def emit(t, w, xt, partner):
            dly = 0.5 * (xt + partner)
            tr = w * (1.0 / 25.0)
            hf = xt - 0.5 * tr - 0.5 * dly
            idx = loff + t
            plsc.store_scatter(trb.at[slot], [idx], tr)
            plsc.store_scatter(dab.at[slot], [idx], dly)
            plsc.store_scatter(hfb.at[slot], [idx], hf)

        w = jnp.zeros((16,), jnp.float32)
        for s in range(HALF_W):
            w = w + g(s)

        def phase(add, sub, poff):
            def body(t, w):
                if add:
                    w = w + g(t + HALF_W)
                if sub:
                    w = w - g(t - HALF_W - 1)
                xt = g(t)
                emit(t, w, xt, g(t + poff))
                return w
            return body

        w = lax.fori_loop(0, HALF_W + 1, phase(True, False, PERIOD), w,
                          unroll=True)
        w = lax.fori_loop(HALF_W + 1, PERIOD, phase(True, True, PERIOD), w,
                          unroll=UNROLL)
        w = lax.fori_loop(PERIOD, T - HALF_W, phase(True, True, -PERIOD), w,
                          unroll=UNROLL)
        w = lax.fori_loop(T - HALF_W, T, phase(False, True, -PERIOD), w,
                          unroll=True)

    in_copy(0, 0).start()

    def group_body(i, carry):
        slot = lax.rem(i, 2)

        @pl.when(i + 1 < ngroups)
        def _():
            in_copy(i + 1, 1 - slot).start()

        in_copy(i, slot).wait()

        @pl.when(i >= 2)
        def _():
            for c in out_copies(i - 2, slot):
                c.wait()

        compute(slot)
        for c in out_copies(i, slot):
            c.start()
        return carry

    lax.fori_loop(0, ngroups, group_body, 0)

    for c in out_copies(ngroups - 2, lax.rem(ngroups - 2, 2)):
        c.wait()
    for c in out_copies(ngroups - 1, lax.rem(ngroups - 1, 2)):
        c.wait()


@jax.jit
def _run(xf):
    out = jax.ShapeDtypeStruct((R * T,), jnp.float32)
    mesh = plsc.VectorSubcoreMesh(core_axis_name="c", subcore_axis_name="s",
                                  num_cores=NC, num_subcores=NS)
    f = pl.kernel(
        _sc_body,
        out_type=(out, out, out),
        mesh=mesh,
        scratch_types=[
            pltpu.VMEM((2, CHUNK), jnp.float32),
            pltpu.VMEM((2, CHUNK), jnp.float32),
            pltpu.VMEM((2, CHUNK), jnp.float32),
            pltpu.VMEM((2, CHUNK), jnp.float32),
            pltpu.SemaphoreType.DMA((2,)),
            pltpu.SemaphoreType.DMA((2,)),
        ],
        compiler_params=pltpu.CompilerParams(use_tc_tiling_on_sc=False,
                                             needs_layout_passes=False),
    )
    return f(xf)


def kernel(x):
    shape = x.shape
    hf, da, tr = _run(x.reshape(-1))
    return hf.reshape(shape), da.reshape(shape), tr.reshape(shape)


# speedup vs baseline: 1.8587x; 1.7851x over previous
import jax
import jax.numpy as jnp
from jax import lax
from jax.experimental import pallas as pl
from jax.experimental.pallas import tpu as pltpu
from jax.experimental.pallas import tpu_sc as plsc

T = 288
PERIOD = 144
HALF_W = 12
NVREG = T // 16
NC, NS, LANES = 2, 16, 16
R = 83200
G_HI, G_LO = 163, 162
ROWS_HI = G_HI * LANES
ROWS_LO = G_LO * LANES
CHUNK = LANES * T
CS_STRIDE = 320


def _sc_body(x_hbm, hf_hbm, da_hbm, tr_hbm, xb, hfb, dab, trb, csb,
             in_sems, out_sems):
    wid = lax.axis_index("s") * NC + lax.axis_index("c")
    base_row = jnp.where(wid < 16, wid * ROWS_HI,
                         16 * ROWS_HI + (wid - 16) * ROWS_LO)
    ngroups = jnp.where(wid < 16, G_HI, G_LO)

    def ebase(i):
        return (base_row + i * LANES) * T

    def in_copy(i, slot):
        return pltpu.make_async_copy(
            x_hbm.at[pl.ds(ebase(i), CHUNK)], xb.at[slot], in_sems.at[slot])

    def out_copies(i, slot):
        sl = pl.ds(ebase(i), CHUNK)
        return (
            pltpu.make_async_copy(hfb.at[slot], hf_hbm.at[sl], out_sems.at[slot]),
            pltpu.make_async_copy(dab.at[slot], da_hbm.at[sl], out_sems.at[slot]),
            pltpu.make_async_copy(trb.at[slot], tr_hbm.at[sl], out_sems.at[slot]),
        )

    zeros = jnp.zeros((16,), jnp.float32)
    for r in range(LANES):
        csb[pl.ds(r * CS_STRIDE, 16)] = zeros

    def compute(slot):
        xs = xb.at[slot]
        hs = hfb.at[slot]
        das = dab.at[slot]
        trs = trb.at[slot]

        def row_body(r, carry):
            xbase = r * T
            cbase = r * CS_STRIDE
            xv = [xs[pl.ds(xbase + 16 * j, 16)] for j in range(NVREG)]
            c = jnp.float32(0.0)
            for j in range(NVREG):
                cs = plsc.cumsum(xv[j]) + c
                csb[pl.ds(cbase + 13 + 16 * j, 16)] = cs
                c = cs[15]
            csb[pl.ds(cbase + 13 + T, 16)] = jnp.full((16,), c, jnp.float32)
            dly = []
            for j in range(9):
                d = 0.5 * (xv[j] + xv[j + 9])
                dly.append(d)
                das[pl.ds(xbase + 16 * j, 16)] = d
                das[pl.ds(xbase + 16 * (j + 9), 16)] = d
            for j in range(NVREG):
                a = csb[pl.ds(cbase + 25 + 16 * j, 16)]
                b = csb[pl.ds(cbase + 16 * j, 16)]
                tr = (a - b) * (1.0 / 25.0)
                trs[pl.ds(xbase + 16 * j, 16)] = tr
                hs[pl.ds(xbase + 16 * j, 16)] = xv[j] - 0.5 * tr - 0.5 * dly[j % 9]
            return carry

        lax.fori_loop(0, LANES, row_body, 0, unroll=2)

    in_copy(0, 0).start()

    def group_body(i, carry):
        slot = lax.rem(i, 2)

        @pl.when(i + 1 < ngroups)
        def _():
            in_copy(i + 1, 1 - slot).start()

        in_copy(i, slot).wait()

        @pl.when(i >= 2)
        def _():
            for cp in out_copies(i - 2, slot):
                cp.wait()

        compute(slot)
        for cp in out_copies(i, slot):
            cp.start()
        return carry

    lax.fori_loop(0, ngroups, group_body, 0)

    for cp in out_copies(ngroups - 2, lax.rem(ngroups - 2, 2)):
        cp.wait()
    for cp in out_copies(ngroups - 1, lax.rem(ngroups - 1, 2)):
        cp.wait()


@jax.jit
def _run(xf):
    out = jax.ShapeDtypeStruct((R * T,), jnp.float32)
    mesh = plsc.VectorSubcoreMesh(core_axis_name="c", subcore_axis_name="s",
                                  num_cores=NC, num_subcores=NS)
    f = pl.kernel(
        _sc_body,
        out_type=(out, out, out),
        mesh=mesh,
        scratch_types=[
            pltpu.VMEM((2, CHUNK), jnp.float32),
            pltpu.VMEM((2, CHUNK), jnp.float32),
            pltpu.VMEM((2, CHUNK), jnp.float32),
            pltpu.VMEM((2, CHUNK), jnp.float32),
            pltpu.VMEM((LANES * CS_STRIDE,), jnp.float32),
            pltpu.SemaphoreType.DMA((2,)),
            pltpu.SemaphoreType.DMA((2,)),
        ],
        compiler_params=pltpu.CompilerParams(use_tc_tiling_on_sc=False,
                                             needs_layout_passes=False),
    )
    return f(xf)


def kernel(x):
    shape = x.shape
    hf, da, tr = _run(x.reshape(-1))
    return hf.reshape(shape), da.reshape(shape), tr.reshape(shape)


# speedup vs baseline: 4.6597x; 2.5070x over previous
import jax
import jax.numpy as jnp
from jax import lax
from jax.experimental import pallas as pl
from jax.experimental.pallas import tpu as pltpu
from jax.experimental.pallas import tpu_sc as plsc

T = 288
PERIOD = 144
HALF_W = 12
NVREG = T // 16
NC, NS = 2, 16
N_SENS = 325
NFULL = N_SENS // 8
NREM = N_SENS - 8 * NFULL
CBLK = 2
ROWS = CBLK * 8
CS_STRIDE = 320


def _sc_body(x_hbm, hf_hbm, da_hbm, tr_hbm,
             xa_b, ha_b, daa_b, ta_b, xb_b, hb_b, dab_b, tb_b, csb,
             in_a, in_b, out_a, out_b):
    wid = lax.axis_index("s") * NC + lax.axis_index("c")
    bidx = wid // 4
    cb = (wid % 4) * 8

    sets = (
        (xa_b, ha_b, daa_b, ta_b, in_a, out_a),
        (xb_b, hb_b, dab_b, tb_b, in_b, out_b),
    )

    def in_full(pb, rt, s):
        src = x_hbm.at[bidx, pl.ds(cb + pb * CBLK, CBLK), pl.ds(8 * rt, 8), :]
        return pltpu.make_async_copy(src, s[0], s[4])

    def in_short(pb, s):
        src = x_hbm.at[bidx, pl.ds(cb + pb * CBLK, CBLK),
                       pl.ds(8 * NFULL, NREM), :]
        return pltpu.make_async_copy(src, s[0].at[:, pl.ds(0, NREM), :], s[4])

    def out_full(pb, rt, s):
        csl = pl.ds(cb + pb * CBLK, CBLK)
        nsl = pl.ds(8 * rt, 8)
        return (
            pltpu.make_async_copy(s[1], hf_hbm.at[bidx, csl, nsl, :], s[5]),
            pltpu.make_async_copy(s[2], da_hbm.at[bidx, csl, nsl, :], s[5]),
            pltpu.make_async_copy(s[3], tr_hbm.at[bidx, csl, nsl, :], s[5]),
        )

    def out_short(pb, s):
        csl = pl.ds(cb + pb * CBLK, CBLK)
        nsl = pl.ds(8 * NFULL, NREM)
        rsl = pl.ds(0, NREM)
        return (
            pltpu.make_async_copy(s[1].at[:, rsl, :], hf_hbm.at[bidx, csl, nsl, :], s[5]),
            pltpu.make_async_copy(s[2].at[:, rsl, :], da_hbm.at[bidx, csl, nsl, :], s[5]),
            pltpu.make_async_copy(s[3].at[:, rsl, :], tr_hbm.at[bidx, csl, nsl, :], s[5]),
        )

    zeros = jnp.zeros((16,), jnp.float32)
    for r in range(ROWS):
        csb[pl.ds(r * CS_STRIDE, 16)] = zeros

    def compute(s):
        xs, hs, das, trs = s[0], s[1], s[2], s[3]

        def row_body(ri, carry):
            for ci in range(CBLK):
                cbase = (ci * 8 + ri) * CS_STRIDE
                xv = [xs[ci, ri, pl.ds(16 * j, 16)] for j in range(NVREG)]
                c = jnp.float32(0.0)
                for j in range(NVREG):
                    ls = plsc.cumsum(xv[j])
                    csb[pl.ds(cbase + 13 + 16 * j, 16)] = ls + c
                    c = c + ls[15]
                csb[pl.ds(cbase + 13 + T, 16)] = jnp.full((16,), c, jnp.float32)
                dly = []
                for j in range(9):
                    d = 0.5 * (xv[j] + xv[j + 9])
                    dly.append(d)
                    das[ci, ri, pl.ds(16 * j, 16)] = d
                    das[ci, ri, pl.ds(16 * (j + 9), 16)] = d
                for j in range(NVREG):
                    a = csb[pl.ds(cbase + 25 + 16 * j, 16)]
                    b = csb[pl.ds(cbase + 16 * j, 16)]
                    tr = (a - b) * (1.0 / 25.0)
                    trs[ci, ri, pl.ds(16 * j, 16)] = tr
                    hs[ci, ri, pl.ds(16 * j, 16)] = (
                        xv[j] - 0.5 * tr - 0.5 * dly[j % 9])
            return carry

        lax.fori_loop(0, 8, row_body, 0)

    def pb_body(pb, carry):
        in_full(pb, 0, sets[0]).start()

        def pair(j, carry2):
            g0 = 2 * j
            g1 = 2 * j + 1
            in_full(pb, g1, sets[1]).start()

            in_full(pb, g0, sets[0]).wait()

            @pl.when(g0 >= 2)
            def _():
                for cp in out_full(pb, g0 - 2, sets[0]):
                    cp.wait()

            compute(sets[0])
            for cp in out_full(pb, g0, sets[0]):
                cp.start()

            @pl.when(g0 + 2 < NFULL)
            def _():
                in_full(pb, g0 + 2, sets[0]).start()

            in_full(pb, g1, sets[1]).wait()

            @pl.when(g1 >= 2)
            def _():
                for cp in out_full(pb, g1 - 2, sets[1]):
                    cp.wait()

            compute(sets[1])
            for cp in out_full(pb, g1, sets[1]):
                cp.start()
            return carry2

        lax.fori_loop(0, NFULL // 2, pair, 0)

        in_short(pb, sets[0]).start()
        in_short(pb, sets[0]).wait()
        for cp in out_full(pb, NFULL - 2, sets[0]):
            cp.wait()
        compute(sets[0])
        for cp in out_short(pb, sets[0]):
            cp.start()

        for cp in out_short(pb, sets[0]):
            cp.wait()
        for cp in out_full(pb, NFULL - 1, sets[1]):
            cp.wait()
        return carry

    lax.fori_loop(0, 8 // CBLK, pb_body, 0)


@jax.jit
def _run(x):
    out = jax.ShapeDtypeStruct(x.shape, jnp.float32)
    mesh = plsc.VectorSubcoreMesh(core_axis_name="c", subcore_axis_name="s",
                                  num_cores=NC, num_subcores=NS)
    buf = pltpu.VMEM((CBLK, 8, T), jnp.float32)
    f = pl.kernel(
        _sc_body,
        out_type=(out, out, out),
        mesh=mesh,
        scratch_types=[
            buf, buf, buf, buf, buf, buf, buf, buf,
            pltpu.VMEM((ROWS * CS_STRIDE,), jnp.float32),
            pltpu.SemaphoreType.DMA,
            pltpu.SemaphoreType.DMA,
            pltpu.SemaphoreType.DMA,
            pltpu.SemaphoreType.DMA,
        ],
        compiler_params=pltpu.CompilerParams(use_tc_tiling_on_sc=True,
                                             needs_layout_passes=False),
    )
    return f(x)


def kernel(x):
    return _run(x)


# speedup vs baseline: 4.7010x; 1.0089x over previous
import jax
import jax.numpy as jnp
from jax import lax
from jax.experimental import pallas as pl
from jax.experimental.pallas import tpu as pltpu
from jax.experimental.pallas import tpu_sc as plsc

T = 288
PERIOD = 144
HALF_W = 12
NVREG = T // 16
NC, NS = 2, 16
N_SENS = 325
NFULL = N_SENS // 8
NREM = N_SENS - 8 * NFULL
CBLK = 2
ROWS = CBLK * 8
CS_STRIDE = 320


def _sc_body(x_hbm, hf_hbm, da_hbm, tr_hbm,
             xa_b, ha_b, daa_b, ta_b, xb_b, hb_b, dab_b, tb_b, csb, tbuf,
             in_a, in_b, out_a, out_b):
    wid = lax.axis_index("s") * NC + lax.axis_index("c")
    bidx = wid // 4
    cb = (wid % 4) * 8

    sets = (
        (xa_b, ha_b, daa_b, ta_b, in_a, out_a),
        (xb_b, hb_b, dab_b, tb_b, in_b, out_b),
    )

    def in_full(pb, rt, s):
        src = x_hbm.at[bidx, pl.ds(cb + pb * CBLK, CBLK), pl.ds(8 * rt, 8), :]
        return pltpu.make_async_copy(src, s[0], s[4])

    def in_short(pb, s):
        src = x_hbm.at[bidx, pl.ds(cb + pb * CBLK, CBLK),
                       pl.ds(8 * NFULL, NREM), :]
        return pltpu.make_async_copy(src, s[0].at[:, pl.ds(0, NREM), :], s[4])

    def out_full(pb, rt, s):
        csl = pl.ds(cb + pb * CBLK, CBLK)
        nsl = pl.ds(8 * rt, 8)
        return (
            pltpu.make_async_copy(s[1], hf_hbm.at[bidx, csl, nsl, :], s[5]),
            pltpu.make_async_copy(s[2], da_hbm.at[bidx, csl, nsl, :], s[5]),
            pltpu.make_async_copy(s[3], tr_hbm.at[bidx, csl, nsl, :], s[5]),
        )

    def out_short(pb, s):
        csl = pl.ds(cb + pb * CBLK, CBLK)
        nsl = pl.ds(8 * NFULL, NREM)
        rsl = pl.ds(0, NREM)
        return (
            pltpu.make_async_copy(s[1].at[:, rsl, :], hf_hbm.at[bidx, csl, nsl, :], s[5]),
            pltpu.make_async_copy(s[2].at[:, rsl, :], da_hbm.at[bidx, csl, nsl, :], s[5]),
            pltpu.make_async_copy(s[3].at[:, rsl, :], tr_hbm.at[bidx, csl, nsl, :], s[5]),
        )

    zeros = jnp.zeros((16,), jnp.float32)
    for r in range(ROWS):
        csb[pl.ds(r * CS_STRIDE, 16)] = zeros

    lane = lax.iota(jnp.int32, 16)
    m15 = lane == 15

    def bcast(v, k):
        return v.at[jnp.full((16,), k, jnp.int32)].get(
            mode="promise_in_bounds")

    def compute(s):
        xs, hs, das, trs = s[0], s[1], s[2], s[3]

        def row_body(ri, carry):
            for ci in range(CBLK):
                cbase = (ci * 8 + ri) * CS_STRIDE
                tbase = (ci * 8 + ri) * 32
                xv = [xs[ci, ri, pl.ds(16 * j, 16)] for j in range(NVREG)]
                dly = []
                for j in range(9):
                    d = 0.5 * (xv[j] + xv[j + 9])
                    dly.append(d)
                    das[ci, ri, pl.ds(16 * j, 16)] = d
                    das[ci, ri, pl.ds(16 * (j + 9), 16)] = d
                ls = []
                for j in range(NVREG):
                    lsj = plsc.cumsum(xv[j])
                    ls.append(lsj)
                    plsc.store_scatter(
                        tbuf, [jnp.full((16,), tbase + j, jnp.int32)], lsj,
                        mask=m15)
                s0 = plsc.cumsum(tbuf[pl.ds(tbase, 16)])
                s1 = plsc.cumsum(tbuf[pl.ds(tbase + 16, 16)]) + bcast(s0, 15)
                for j in range(NVREG):
                    if j == 0:
                        cs = ls[0]
                    elif j <= 16:
                        cs = ls[j] + bcast(s0, j - 1)
                    else:
                        cs = ls[j] + bcast(s1, j - 17)
                    csb[pl.ds(cbase + 13 + 16 * j, 16)] = cs
                csb[pl.ds(cbase + 13 + T, 16)] = bcast(s1, 1)
                for j in range(NVREG):
                    a = csb[pl.ds(cbase + 25 + 16 * j, 16)]
                    b = csb[pl.ds(cbase + 16 * j, 16)]
                    tr = (a - b) * (1.0 / 25.0)
                    trs[ci, ri, pl.ds(16 * j, 16)] = tr
                    hs[ci, ri, pl.ds(16 * j, 16)] = (
                        xv[j] - 0.5 * tr - 0.5 * dly[j % 9])
            return carry

        lax.fori_loop(0, 8, row_body, 0)

    def pb_body(pb, carry):
        in_full(pb, 0, sets[0]).start()

        def pair(j, carry2):
            g0 = 2 * j
            g1 = 2 * j + 1
            in_full(pb, g1, sets[1]).start()

            in_full(pb, g0, sets[0]).wait()

            @pl.when(g0 >= 2)
            def _():
                for cp in out_full(pb, g0 - 2, sets[0]):
                    cp.wait()

            compute(sets[0])
            for cp in out_full(pb, g0, sets[0]):
                cp.start()

            @pl.when(g0 + 2 < NFULL)
            def _():
                in_full(pb, g0 + 2, sets[0]).start()

            in_full(pb, g1, sets[1]).wait()

            @pl.when(g1 >= 2)
            def _():
                for cp in out_full(pb, g1 - 2, sets[1]):
                    cp.wait()

            compute(sets[1])
            for cp in out_full(pb, g1, sets[1]):
                cp.start()
            return carry2

        lax.fori_loop(0, NFULL // 2, pair, 0)

        in_short(pb, sets[0]).start()
        in_short(pb, sets[0]).wait()
        for cp in out_full(pb, NFULL - 2, sets[0]):
            cp.wait()
        compute(sets[0])
        for cp in out_short(pb, sets[0]):
            cp.start()

        for cp in out_short(pb, sets[0]):
            cp.wait()
        for cp in out_full(pb, NFULL - 1, sets[1]):
            cp.wait()
        return carry

    lax.fori_loop(0, 8 // CBLK, pb_body, 0)


@jax.jit
def _run(x):
    out = jax.ShapeDtypeStruct(x.shape, jnp.float32)
    mesh = plsc.VectorSubcoreMesh(core_axis_name="c", subcore_axis_name="s",
                                  num_cores=NC, num_subcores=NS)
    buf = pltpu.VMEM((CBLK, 8, T), jnp.float32)
    f = pl.kernel(
        _sc_body,
        out_type=(out, out, out),
        mesh=mesh,
        scratch_types=[
            buf, buf, buf, buf, buf, buf, buf, buf,
            pltpu.VMEM((ROWS * CS_STRIDE,), jnp.float32),
            pltpu.VMEM((ROWS * 32,), jnp.float32),
            pltpu.SemaphoreType.DMA,
            pltpu.SemaphoreType.DMA,
            pltpu.SemaphoreType.DMA,
            pltpu.SemaphoreType.DMA,
        ],
        compiler_params=pltpu.CompilerParams(use_tc_tiling_on_sc=True,
                                             needs_layout_passes=False),
    )
    return f(x)


def kernel(x):
    return _run(x)


# speedup vs baseline: 5.3705x; 1.1424x over previous
import jax
import jax.numpy as jnp
from jax import lax
from jax.experimental import pallas as pl
from jax.experimental.pallas import tpu as pltpu
from jax.experimental.pallas import tpu_sc as plsc

T = 288
PERIOD = 144
HALF_W = 12
NC, NS = 2, 16
N_SENS = 325
NB = 128
NTAIL = N_SENS - 2 * NB
PT = 48
NPH = T // PT
NU = NB // 16


def _sc_body(x_hbm, xtail_hbm, hf_hbm, da_hbm, tr_hbm,
             xa, xb, oh_a, od_a, ot_a, oh_b, od_b, ot_b, wscr,
             in_a, in_b, out_a, out_b):
    wid = lax.axis_index("s") * NC + lax.axis_index("c")
    plane0 = wid * 8

    def bc(p):
        return p // 32, p % 32

    def in_full(p, n0, buf, sem):
        b, c = bc(p)
        return pltpu.make_async_copy(
            x_hbm.at[b, c, :, pl.ds(n0, NB)], buf, sem)

    def in_tail(p, buf, sem):
        b, c = bc(p)
        return pltpu.make_async_copy(xtail_hbm.at[b, c], buf, sem)

    def out_full(p, n0, ph, obufs, sem):
        b, c = bc(p)
        tsl = pl.ds(PT * ph, PT)
        nsl = pl.ds(n0, NB)
        oh, od, ot = obufs
        return (
            pltpu.make_async_copy(oh, hf_hbm.at[b, c, tsl, nsl], sem),
            pltpu.make_async_copy(od, da_hbm.at[b, c, tsl, nsl], sem),
            pltpu.make_async_copy(ot, tr_hbm.at[b, c, tsl, nsl], sem),
        )

    def out_tail(p, ph, obufs, sem):
        return out_full(p, 2 * NB, ph, obufs, sem)

    osets = ((oh_a, od_a, ot_a), (oh_b, od_b, ot_b))
    osems = (out_a, out_b)

    def compute(xbuf, out_desc, wait_first):
        def initu(u, carry):
            def it(s, acc):
                return acc + xbuf[s, pl.ds(u * 16, 16)]
            wscr[pl.ds(u * 16, 16)] = lax.fori_loop(
                0, HALF_W, it, jnp.zeros((16,), jnp.float32))
            return carry

        lax.fori_loop(0, NU, initu, 0)

        for ph in range(NPH):
            obufs = osets[ph % 2]
            sem = osems[ph % 2]
            if ph < 2 and wait_first is not None:
                @pl.when(wait_first)
                def _():
                    for cp in out_desc(ph, obufs, sem):
                        cp.wait()
            else:
                for cp in out_desc(ph, obufs, sem):
                    cp.wait()
            oh, od, ot = obufs
            t0 = PT * ph

            def uloop(u, carry):
                def tloop(t, w):
                    col = pl.ds(u * 16, 16)
                    x_add = xbuf[jnp.minimum(t + HALF_W, T - 1), col]
                    w = w + jnp.where(t + HALF_W <= T - 1, x_add, 0.0)
                    x_sub = xbuf[jnp.maximum(t - HALF_W - 1, 0), col]
                    w = w - jnp.where(t - HALF_W - 1 >= 0, x_sub, 0.0)
                    xt = xbuf[t, col]
                    po = jnp.where(t < PERIOD, t + PERIOD, t - PERIOD)
                    xp = xbuf[po, col]
                    dly = 0.5 * (xt + xp)
                    tr = w * (1.0 / 25.0)
                    ot[t - t0, col] = tr
                    od[t - t0, col] = dly
                    oh[t - t0, col] = xt - 0.5 * tr - 0.5 * dly
                    return w

                w = lax.fori_loop(t0, t0 + PT, tloop,
                                  wscr[pl.ds(u * 16, 16)])
                wscr[pl.ds(u * 16, 16)] = w
                return carry

            lax.fori_loop(0, NU, uloop, 0)
            for cp in out_desc(ph, obufs, sem):
                cp.start()

    in_full(plane0, 0, xa, in_a).start()

    def full_body(j, carry):
        p = plane0 + j
        in_full(p, NB, xb, in_b).start()
        in_full(p, 0, xa, in_a).wait()
        compute(xa, lambda ph, ob, sm: out_full(p, 0, ph, ob, sm),
                wait_first=(j > 0))

        @pl.when(j < 7)
        def _():
            in_full(p + 1, 0, xa, in_a).start()

        in_full(p, NB, xb, in_b).wait()
        compute(xb, lambda ph, ob, sm: out_full(p, NB, ph, ob, sm),
                wait_first=None)
        return carry

    lax.fori_loop(0, 8, full_body, 0)

    for cp in out_full(plane0, 0, 0, osets[0], osems[0]):
        cp.wait()
    for cp in out_full(plane0, 0, 1, osets[1], osems[1]):
        cp.wait()

    in_tail(plane0, xa, in_a).start()

    def tail_body(j, carry):
        p0 = plane0 + 2 * j
        in_tail(p0 + 1, xb, in_b).start()
        in_tail(p0, xa, in_a).wait()
        compute(xa, lambda ph, ob, sm: out_tail(p0, ph, ob, sm),
                wait_first=(j > 0))

        @pl.when(j < 3)
        def _():
            in_tail(p0 + 2, xa, in_a).start()

        in_tail(p0 + 1, xb, in_b).wait()
        compute(xb, lambda ph, ob, sm: out_tail(p0 + 1, ph, ob, sm),
                wait_first=None)
        return carry

    lax.fori_loop(0, 4, tail_body, 0)

    for cp in out_tail(plane0, 0, osets[0], osems[0]):
        cp.wait()
    for cp in out_tail(plane0, 1, osets[1], osems[1]):
        cp.wait()


@jax.jit
def _run(xt, xtail):
    b, c, t, _ = xt.shape
    out = jax.ShapeDtypeStruct((b, c, t, 3 * NB), jnp.float32)
    mesh = plsc.VectorSubcoreMesh(core_axis_name="c", subcore_axis_name="s",
                                  num_cores=NC, num_subcores=NS)
    inbuf = pltpu.VMEM((T, NB), jnp.float32)
    obuf = pltpu.VMEM((PT, NB), jnp.float32)
    f = pl.kernel(
        _sc_body,
        out_type=(out, out, out),
        mesh=mesh,
        scratch_types=[
            inbuf, inbuf,
            obuf, obuf, obuf, obuf, obuf, obuf,
            pltpu.VMEM((NB,), jnp.float32),
            pltpu.SemaphoreType.DMA,
            pltpu.SemaphoreType.DMA,
            pltpu.SemaphoreType.DMA,
            pltpu.SemaphoreType.DMA,
        ],
        compiler_params=pltpu.CompilerParams(use_tc_tiling_on_sc=True,
                                             needs_layout_passes=False),
    )
    return f(xt, xtail)


def kernel(x):
    xt = jnp.transpose(x, (0, 1, 3, 2))
    xtail = jnp.pad(xt[..., 2 * NB:], ((0, 0), (0, 0), (0, 0), (0, NB - NTAIL)))
    hf, da, tr = _run(xt, xtail)
    perm = (0, 1, 3, 2)
    return (jnp.transpose(hf[..., :N_SENS], perm),
            jnp.transpose(da[..., :N_SENS], perm),
            jnp.transpose(tr[..., :N_SENS], perm))


# speedup vs baseline: 8.2035x; 1.5275x over previous
import jax
import jax.numpy as jnp
from jax import lax
from jax.experimental import pallas as pl
from jax.experimental.pallas import tpu as pltpu
from jax.experimental.pallas import tpu_sc as plsc

T = 288
PERIOD = 144
HALF_W = 12
NC, NS = 2, 16
N_SENS = 325
NB = 128
NTAIL = N_SENS - 2 * NB
PT = 48
NPH = T // PT
NU = NB // 16


def _sc_body(x_hbm, xtail_hbm, hf_hbm, da_hbm, tr_hbm,
             xa, xb, oh_a, od_a, ot_a, oh_b, od_b, ot_b, wscr,
             in_a, in_b, out_a, out_b):
    wid = lax.axis_index("s") * NC + lax.axis_index("c")
    plane0 = wid * 8

    def bc(p):
        return p // 32, p % 32

    def in_full(p, n0, buf, sem):
        b, c = bc(p)
        return pltpu.make_async_copy(
            x_hbm.at[b, c, :, pl.ds(n0, NB)], buf, sem)

    def in_tail(p, buf, sem):
        b, c = bc(p)
        return pltpu.make_async_copy(xtail_hbm.at[b, c], buf, sem)

    def out_full(p, n0, ph, obufs, sem):
        b, c = bc(p)
        tsl = pl.ds(PT * ph, PT)
        nsl = pl.ds(n0, NB)
        oh, od, ot = obufs
        return (
            pltpu.make_async_copy(oh, hf_hbm.at[b, c, tsl, nsl], sem),
            pltpu.make_async_copy(od, da_hbm.at[b, c, tsl, nsl], sem),
            pltpu.make_async_copy(ot, tr_hbm.at[b, c, tsl, nsl], sem),
        )

    def out_tail(p, ph, obufs, sem):
        return out_full(p, 2 * NB, ph, obufs, sem)

    osets = ((oh_a, od_a, ot_a), (oh_b, od_b, ot_b))
    osems = (out_a, out_b)

    def compute(xbuf, out_desc, wait_first):
        def initu(u, carry):
            def it(s, acc):
                return acc + xbuf[s, pl.ds(u * 16, 16)]
            wscr[pl.ds(u * 16, 16)] = lax.fori_loop(
                0, HALF_W, it, jnp.zeros((16,), jnp.float32))
            return carry

        lax.fori_loop(0, NU, initu, 0)

        for ph in range(NPH):
            obufs = osets[ph % 2]
            sem = osems[ph % 2]
            if ph < 2 and wait_first is not None:
                @pl.when(wait_first)
                def _():
                    for cp in out_desc(ph, obufs, sem):
                        cp.wait()
            else:
                for cp in out_desc(ph, obufs, sem):
                    cp.wait()
            oh, od, ot = obufs
            t0 = PT * ph

            def uloop(u, carry):
                def tloop(t, w):
                    col = pl.ds(u * 16, 16)
                    x_add = xbuf[jnp.minimum(t + HALF_W, T - 1), col]
                    w = w + jnp.where(t + HALF_W <= T - 1, x_add, 0.0)
                    x_sub = xbuf[jnp.maximum(t - HALF_W - 1, 0), col]
                    w = w - jnp.where(t - HALF_W - 1 >= 0, x_sub, 0.0)
                    xt = xbuf[t, col]
                    po = jnp.where(t < PERIOD, t + PERIOD, t - PERIOD)
                    xp = xbuf[po, col]
                    dly = 0.5 * (xt + xp)
                    tr = w * (1.0 / 25.0)
                    ot[t - t0, col] = tr
                    od[t - t0, col] = dly
                    oh[t - t0, col] = xt - 0.5 * tr - 0.5 * dly
                    return w

                w = lax.fori_loop(t0, t0 + PT, tloop,
                                  wscr[pl.ds(u * 16, 16)], unroll=4)
                wscr[pl.ds(u * 16, 16)] = w
                return carry

            lax.fori_loop(0, NU, uloop, 0)
            for cp in out_desc(ph, obufs, sem):
                cp.start()

    in_full(plane0, 0, xa, in_a).start()

    def full_body(j, carry):
        p = plane0 + j
        in_full(p, NB, xb, in_b).start()
        in_full(p, 0, xa, in_a).wait()
        compute(xa, lambda ph, ob, sm: out_full(p, 0, ph, ob, sm),
                wait_first=(j > 0))

        @pl.when(j < 7)
        def _():
            in_full(p + 1, 0, xa, in_a).start()

        in_full(p, NB, xb, in_b).wait()
        compute(xb, lambda ph, ob, sm: out_full(p, NB, ph, ob, sm),
                wait_first=None)
        return carry

    lax.fori_loop(0, 8, full_body, 0)

    for cp in out_full(plane0, 0, 0, osets[0], osems[0]):
        cp.wait()
    for cp in out_full(plane0, 0, 1, osets[1], osems[1]):
        cp.wait()

    in_tail(plane0, xa, in_a).start()

    def tail_body(j, carry):
        p0 = plane0 + 2 * j
        in_tail(p0 + 1, xb, in_b).start()
        in_tail(p0, xa, in_a).wait()
        compute(xa, lambda ph, ob, sm: out_tail(p0, ph, ob, sm),
                wait_first=(j > 0))

        @pl.when(j < 3)
        def _():
            in_tail(p0 + 2, xa, in_a).start()

        in_tail(p0 + 1, xb, in_b).wait()
        compute(xb, lambda ph, ob, sm: out_tail(p0 + 1, ph, ob, sm),
                wait_first=None)
        return carry

    lax.fori_loop(0, 4, tail_body, 0)

    for cp in out_tail(plane0, 0, osets[0], osems[0]):
        cp.wait()
    for cp in out_tail(plane0, 1, osets[1], osems[1]):
        cp.wait()


@jax.jit
def _run(xt, xtail):
    b, c, t, _ = xt.shape
    out = jax.ShapeDtypeStruct((b, c, t, 3 * NB), jnp.float32)
    mesh = plsc.VectorSubcoreMesh(core_axis_name="c", subcore_axis_name="s",
                                  num_cores=NC, num_subcores=NS)
    inbuf = pltpu.VMEM((T, NB), jnp.float32)
    obuf = pltpu.VMEM((PT, NB), jnp.float32)
    f = pl.kernel(
        _sc_body,
        out_type=(out, out, out),
        mesh=mesh,
        scratch_types=[
            inbuf, inbuf,
            obuf, obuf, obuf, obuf, obuf, obuf,
            pltpu.VMEM((NB,), jnp.float32),
            pltpu.SemaphoreType.DMA,
            pltpu.SemaphoreType.DMA,
            pltpu.SemaphoreType.DMA,
            pltpu.SemaphoreType.DMA,
        ],
        compiler_params=pltpu.CompilerParams(use_tc_tiling_on_sc=True,
                                             needs_layout_passes=False),
    )
    return f(xt, xtail)


def kernel(x):
    xt = jnp.transpose(x, (0, 1, 3, 2))
    xtail = jnp.pad(xt[..., 2 * NB:], ((0, 0), (0, 0), (0, 0), (0, NB - NTAIL)))
    hf, da, tr = _run(xt, xtail)
    perm = (0, 1, 3, 2)
    return (jnp.transpose(hf[..., :N_SENS], perm),
            jnp.transpose(da[..., :N_SENS], perm),
            jnp.transpose(tr[..., :N_SENS], perm))


# speedup vs baseline: 8.6335x; 1.0524x over previous
import jax
import jax.numpy as jnp
from jax import lax
from jax.experimental import pallas as pl
from jax.experimental.pallas import tpu as pltpu
from jax.experimental.pallas import tpu_sc as plsc

T = 288
PERIOD = 144
HALF_W = 12
NC, NS = 2, 16
N_SENS = 325
NB = 128
NTAIL = N_SENS - 2 * NB
PT = 48
NPH = T // PT
NU = NB // 16


def _sc_body(x_hbm, xtail_hbm, hf_hbm, da_hbm, tr_hbm,
             xa, xb, oh_a, od_a, ot_a, oh_b, od_b, ot_b, wscr,
             in_a, in_b, out_a, out_b):
    wid = lax.axis_index("s") * NC + lax.axis_index("c")
    plane0 = wid * 8

    def bc(p):
        return p // 32, p % 32

    def in_full(p, n0, buf, sem):
        b, c = bc(p)
        return pltpu.make_async_copy(
            x_hbm.at[b, c, :, pl.ds(n0, NB)], buf, sem)

    def in_tail(p, buf, sem):
        b, c = bc(p)
        return pltpu.make_async_copy(xtail_hbm.at[b, c], buf, sem)

    def out_full(p, n0, ph, obufs, sem):
        b, c = bc(p)
        tsl = pl.ds(PT * ph, PT)
        nsl = pl.ds(n0, NB)
        oh, od, ot = obufs
        return (
            pltpu.make_async_copy(oh, hf_hbm.at[b, c, tsl, nsl], sem),
            pltpu.make_async_copy(od, da_hbm.at[b, c, tsl, nsl], sem),
            pltpu.make_async_copy(ot, tr_hbm.at[b, c, tsl, nsl], sem),
        )

    def out_tail(p, ph, obufs, sem):
        return out_full(p, 2 * NB, ph, obufs, sem)

    osets = ((oh_a, od_a, ot_a), (oh_b, od_b, ot_b))
    osems = (out_a, out_b)

    def compute(xbuf, out_desc, wait_first):
        def initu(u, carry):
            def it(s, acc):
                return acc + xbuf[s, pl.ds(u * 16, 16)]
            wscr[pl.ds(u * 16, 16)] = lax.fori_loop(
                0, HALF_W, it, jnp.zeros((16,), jnp.float32))
            return carry

        lax.fori_loop(0, NU, initu, 0)

        for ph in range(NPH):
            obufs = osets[ph % 2]
            sem = osems[ph % 2]
            if ph < 2 and wait_first is not None:
                @pl.when(wait_first)
                def _():
                    for cp in out_desc(ph, obufs, sem):
                        cp.wait()
            else:
                for cp in out_desc(ph, obufs, sem):
                    cp.wait()
            oh, od, ot = obufs
            t0 = PT * ph

            def uloop(u, carry):
                def tloop(t, w):
                    col = pl.ds(u * 16, 16)
                    x_add = xbuf[jnp.minimum(t + HALF_W, T - 1), col]
                    w = w + jnp.where(t + HALF_W <= T - 1, x_add, 0.0)
                    x_sub = xbuf[jnp.maximum(t - HALF_W - 1, 0), col]
                    w = w - jnp.where(t - HALF_W - 1 >= 0, x_sub, 0.0)
                    xt = xbuf[t, col]
                    po = jnp.where(t < PERIOD, t + PERIOD, t - PERIOD)
                    xp = xbuf[po, col]
                    dly = 0.5 * (xt + xp)
                    tr = w * (1.0 / 25.0)
                    ot[t - t0, col] = tr
                    od[t - t0, col] = dly
                    oh[t - t0, col] = xt - 0.5 * tr - 0.5 * dly
                    return w

                w = lax.fori_loop(t0, t0 + PT, tloop,
                                  wscr[pl.ds(u * 16, 16)], unroll=8)
                wscr[pl.ds(u * 16, 16)] = w
                return carry

            lax.fori_loop(0, NU, uloop, 0)
            for cp in out_desc(ph, obufs, sem):
                cp.start()

    in_full(plane0, 0, xa, in_a).start()

    def full_body(j, carry):
        p = plane0 + j
        in_full(p, NB, xb, in_b).start()
        in_full(p, 0, xa, in_a).wait()
        compute(xa, lambda ph, ob, sm: out_full(p, 0, ph, ob, sm),
                wait_first=(j > 0))

        @pl.when(j < 7)
        def _():
            in_full(p + 1, 0, xa, in_a).start()

        in_full(p, NB, xb, in_b).wait()
        compute(xb, lambda ph, ob, sm: out_full(p, NB, ph, ob, sm),
                wait_first=None)
        return carry

    lax.fori_loop(0, 8, full_body, 0)

    for cp in out_full(plane0, 0, 0, osets[0], osems[0]):
        cp.wait()
    for cp in out_full(plane0, 0, 1, osets[1], osems[1]):
        cp.wait()

    in_tail(plane0, xa, in_a).start()

    def tail_body(j, carry):
        p0 = plane0 + 2 * j
        in_tail(p0 + 1, xb, in_b).start()
        in_tail(p0, xa, in_a).wait()
        compute(xa, lambda ph, ob, sm: out_tail(p0, ph, ob, sm),
                wait_first=(j > 0))

        @pl.when(j < 3)
        def _():
            in_tail(p0 + 2, xa, in_a).start()

        in_tail(p0 + 1, xb, in_b).wait()
        compute(xb, lambda ph, ob, sm: out_tail(p0 + 1, ph, ob, sm),
                wait_first=None)
        return carry

    lax.fori_loop(0, 4, tail_body, 0)

    for cp in out_tail(plane0, 0, osets[0], osems[0]):
        cp.wait()
    for cp in out_tail(plane0, 1, osets[1], osems[1]):
        cp.wait()


@jax.jit
def _run(xt, xtail):
    b, c, t, _ = xt.shape
    out = jax.ShapeDtypeStruct((b, c, t, 3 * NB), jnp.float32)
    mesh = plsc.VectorSubcoreMesh(core_axis_name="c", subcore_axis_name="s",
                                  num_cores=NC, num_subcores=NS)
    inbuf = pltpu.VMEM((T, NB), jnp.float32)
    obuf = pltpu.VMEM((PT, NB), jnp.float32)
    f = pl.kernel(
        _sc_body,
        out_type=(out, out, out),
        mesh=mesh,
        scratch_types=[
            inbuf, inbuf,
            obuf, obuf, obuf, obuf, obuf, obuf,
            pltpu.VMEM((NB,), jnp.float32),
            pltpu.SemaphoreType.DMA,
            pltpu.SemaphoreType.DMA,
            pltpu.SemaphoreType.DMA,
            pltpu.SemaphoreType.DMA,
        ],
        compiler_params=pltpu.CompilerParams(use_tc_tiling_on_sc=True,
                                             needs_layout_passes=False),
    )
    return f(xt, xtail)


def kernel(x):
    xt = jnp.transpose(x, (0, 1, 3, 2))
    xtail = jnp.pad(xt[..., 2 * NB:], ((0, 0), (0, 0), (0, 0), (0, NB - NTAIL)))
    hf, da, tr = _run(xt, xtail)
    perm = (0, 1, 3, 2)
    return (jnp.transpose(hf[..., :N_SENS], perm),
            jnp.transpose(da[..., :N_SENS], perm),
            jnp.transpose(tr[..., :N_SENS], perm))
